# baseline (speedup 1.0000x reference)
import jax
import jax.numpy as jnp
from jax import lax
from jax.experimental import pallas as pl
from jax.experimental.pallas import tpu as pltpu
from jax.experimental.pallas import tpu_sc as plsc

B = 128
T = 2048
V = 29
NUM_CORES = 2
NUM_SUBCORES = 16
NW = NUM_CORES * NUM_SUBCORES
ROWS_PER_W = B // NW
NBT = B // 8
NTT = T // 128
NSC = 3
TCW = (NTT - NSC + 1) * 128
TBLK = 256

_mesh = plsc.VectorSubcoreMesh(
    core_axis_name="c", subcore_axis_name="s",
    num_cores=NUM_CORES, num_subcores=NUM_SUBCORES,
)


def _argmax_tree(xs):
    level = []
    for i in range(V // 2):
        a, b = xs[2 * i], xs[2 * i + 1]
        gt = b > a
        level.append((jnp.where(gt, b, a),
                      jnp.where(gt, jnp.int32(2 * i + 1), jnp.int32(2 * i))))
    level.append((xs[V - 1], jnp.full(xs[0].shape, V - 1, jnp.int32)))
    while len(level) > 1:
        nxt = []
        for i in range(len(level) // 2):
            va, ia = level[2 * i]
            vb, ib = level[2 * i + 1]
            gt = vb > va
            nxt.append((jnp.where(gt, vb, va), jnp.where(gt, ib, ia)))
        if len(level) % 2:
            nxt.append(level[-1])
        level = nxt
    return level[0]


@jax.jit
def _ctc_sc(lp5):
    @pl.kernel(
        out_type=[
            jax.ShapeDtypeStruct((NBT, NSC, 8, 128), jnp.int32),
            jax.ShapeDtypeStruct((NBT, NSC, 8, 128), jnp.int32),
            jax.ShapeDtypeStruct((NBT, NSC, 8, 128), jnp.float32),
        ],
        mesh=_mesh,
        compiler_params=pltpu.CompilerParams(needs_layout_passes=False),
        scratch_types=(
            [pltpu.VMEM((V, 1, 128), jnp.float32) for _ in range(2 * NSC)]
            + [
                pltpu.VMEM((NSC, 128), jnp.int32),
                pltpu.VMEM((NSC, 128), jnp.int32),
                pltpu.VMEM((NSC, 128), jnp.float32),
                pltpu.SemaphoreType.DMA,
                pltpu.SemaphoreType.DMA,
            ]
        ),
    )
    def k(lp_hbm, idx_hbm, val_hbm, mp_hbm, *rest):
        tilebufs = (rest[:NSC], rest[NSC:2 * NSC])
        idxrow, valrow, mprow, sem0, sem1 = rest[2 * NSC:]
        wid = lax.axis_index("s") * NUM_CORES + lax.axis_index("c")
        iota16 = lax.iota(jnp.int32, 16)
        sems = (sem0, sem1)

        def start(step):
            b = wid * ROWS_PER_W + step
            s = step % 2
            return [
                pltpu.async_copy(
                    lp_hbm.at[:, b // 8, pl.ds(tt, 1), b % 8, :],
                    tilebufs[s][tt], sems[s])
                for tt in range(NSC)
            ]

        handles = {0: start(0)}
        for step in range(ROWS_PER_W):
            bufs = tilebufs[step % 2]
            for h in handles.pop(step):
                h.wait()
            if step + 1 < ROWS_PER_W:
                handles[step + 1] = start(step + 1)

            for tt_l in range(NSC):
                buf = bufs[tt_l]

                @plsc.parallel_loop(0, 8, unroll=2)
                def _pass1(g):
                    l0 = g * 16
                    xs = [buf[v, 0, pl.ds(l0, 16)] for v in range(V)]
                    cmax, cidx = _argmax_tree(xs)
                    idxrow[tt_l, pl.ds(l0, 16)] = cidx
                    mprow[tt_l, pl.ds(l0, 16)] = jnp.exp(cmax)

            @plsc.parallel_loop(0, NSC * 8, unroll=2)
            def _pass2(g):
                tt_l = g // 8
                l0 = (g % 8) * 16
                cur = idxrow[tt_l, pl.ds(l0, 16)]
                t = tt_l * 128 + l0 + iota16
                pt = jnp.maximum(t - 1, 0)
                prev = plsc.load_gather(
                    idxrow, [lax.shift_right_logical(pt, 7), pt & 127])
                valid = (cur != 0) & ((cur != prev) | (t == 0))
                valrow[tt_l, pl.ds(l0, 16)] = valid.astype(jnp.int32)

            b = wid * ROWS_PER_W + step
            pltpu.sync_copy(idxrow, idx_hbm.at[b // 8, :, b % 8, :])
            pltpu.sync_copy(valrow, val_hbm.at[b // 8, :, b % 8, :])
            pltpu.sync_copy(mprow, mp_hbm.at[b // 8, :, b % 8, :])

    return k(lp5)


def _tc_body(x_ref, idx_ref, val_ref, mp_ref, prev_ref):
    j = pl.program_id(1)
    cmax, cidx = _argmax_tree([x_ref[v] for v in range(V)])
    idx_ref[...] = cidx
    mp_ref[...] = jnp.exp(cmax)

    @pl.when(j == 0)
    def _():
        prev_ref[...] = jnp.full((8, 128), -1, jnp.int32)

    carry = prev_ref[:, 0:1]
    prev = jnp.concatenate([carry, cidx[:, :-1]], axis=1)
    val_ref[...] = ((cidx != 0) & (cidx != prev)).astype(jnp.int32)
    prev_ref[:, 0:1] = cidx[:, TBLK - 1:TBLK]


@jax.jit
def _ctc_tc(lp3):
    noff = (NSC - 1) * 128 // TBLK
    return pl.pallas_call(
        _tc_body,
        grid=(NBT, TCW // TBLK),
        in_specs=[pl.BlockSpec((V, 8, TBLK), lambda i, j: (0, i, j + noff))],
        out_specs=[
            pl.BlockSpec((8, TBLK), lambda i, j: (i, j)),
            pl.BlockSpec((8, TBLK), lambda i, j: (i, j)),
            pl.BlockSpec((8, TBLK), lambda i, j: (i, j)),
        ],
        out_shape=[
            jax.ShapeDtypeStruct((B, TCW), jnp.int32),
            jax.ShapeDtypeStruct((B, TCW), jnp.int32),
            jax.ShapeDtypeStruct((B, TCW), jnp.float32),
        ],
        scratch_shapes=[pltpu.VMEM((8, 128), jnp.int32)],
        compiler_params=pltpu.CompilerParams(
            dimension_semantics=("arbitrary", "arbitrary")),
    )(lp3)


def kernel(log_probs):
    lp5 = log_probs.reshape(NBT, 8, NTT, 128, V).transpose(4, 0, 2, 1, 3)
    lp3 = log_probs.transpose(2, 0, 1)

    sc_idx4, sc_val4, sc_mp4 = _ctc_sc(lp5)
    tc_idx, tc_val, tc_mp = _ctc_tc(lp3)

    def unview(x4):
        return x4.transpose(0, 2, 1, 3).reshape(B, NSC * 128)

    idx = jnp.concatenate([unview(sc_idx4), tc_idx[:, 128:]], axis=1)
    val = jnp.concatenate([unview(sc_val4), tc_val[:, 128:]], axis=1)
    mp = jnp.concatenate([unview(sc_mp4), tc_mp[:, 128:]], axis=1)
    return idx, val.astype(bool), mp


# speedup vs baseline: 2.3360x; 2.3360x over previous
import jax
import jax.numpy as jnp
from jax import lax
from jax.experimental import pallas as pl
from jax.experimental.pallas import tpu as pltpu
from jax.experimental.pallas import tpu_sc as plsc

B = 128
T = 2048
V = 29
NUM_CORES = 2
NUM_SUBCORES = 16
NW = NUM_CORES * NUM_SUBCORES
NBT = B // 8
NTT = T // 128
NBT_SC = NW // 8
B_SC = 8 * NBT_SC

_mesh = plsc.VectorSubcoreMesh(
    core_axis_name="c", subcore_axis_name="s",
    num_cores=NUM_CORES, num_subcores=NUM_SUBCORES,
)


def _argmax_tree(xs):
    level = []
    for i in range(V // 2):
        a, b = xs[2 * i], xs[2 * i + 1]
        gt = b > a
        level.append((jnp.where(gt, b, a),
                      jnp.where(gt, jnp.int32(2 * i + 1), jnp.int32(2 * i))))
    level.append((xs[V - 1], jnp.full(xs[0].shape, V - 1, jnp.int32)))
    while len(level) > 1:
        nxt = []
        for i in range(len(level) // 2):
            va, ia = level[2 * i]
            vb, ib = level[2 * i + 1]
            gt = vb > va
            nxt.append((jnp.where(gt, vb, va), jnp.where(gt, ib, ia)))
        if len(level) % 2:
            nxt.append(level[-1])
        level = nxt
    return level[0]


@jax.jit
def _ctc_sc(lp5):
    @pl.kernel(
        out_type=[
            jax.ShapeDtypeStruct((NBT_SC, NTT, 8, 128), jnp.int32),
            jax.ShapeDtypeStruct((NBT_SC, NTT, 8, 128), jnp.int32),
            jax.ShapeDtypeStruct((NBT_SC, NTT, 8, 128), jnp.float32),
        ],
        mesh=_mesh,
        compiler_params=pltpu.CompilerParams(needs_layout_passes=False),
        scratch_types=[
            pltpu.VMEM((V, NTT // 2, 128), jnp.float32),
            pltpu.VMEM((V, NTT // 2, 128), jnp.float32),
            pltpu.VMEM((NTT, 128), jnp.int32),
            pltpu.VMEM((NTT, 128), jnp.int32),
            pltpu.VMEM((NTT, 128), jnp.float32),
            pltpu.SemaphoreType.DMA,
            pltpu.SemaphoreType.DMA,
        ],
    )
    def k(lp_hbm, idx_hbm, val_hbm, mp_hbm,
          buf0, buf1, idxrow, valrow, mprow, sem0, sem1):
        wid = lax.axis_index("s") * NUM_CORES + lax.axis_index("c")
        iota16 = lax.iota(jnp.int32, 16)
        bufs = (buf0, buf1)
        sems = (sem0, sem1)
        bt, bs = wid // 8, wid % 8
        half_tt = NTT // 2

        def start(half):
            return pltpu.async_copy(
                lp_hbm.at[:, bt, pl.ds(half * half_tt, half_tt), bs, :],
                bufs[half], sems[half])

        handles = {0: start(0)}
        for half in range(2):
            buf = bufs[half]
            handles.pop(half).wait()
            if half == 0:
                handles[1] = start(1)

            @plsc.parallel_loop(0, half_tt * 8, unroll=2)
            def _pass1(g):
                tt_l = half * half_tt + g // 8
                l0 = (g % 8) * 16
                xs = [buf[v, g // 8, pl.ds(l0, 16)] for v in range(V)]
                cmax, cidx = _argmax_tree(xs)
                idxrow[tt_l, pl.ds(l0, 16)] = cidx
                mprow[tt_l, pl.ds(l0, 16)] = jnp.exp(cmax)

            @plsc.parallel_loop(0, half_tt * 8, unroll=2)
            def _pass2(g):
                tt_l = half * half_tt + g // 8
                l0 = (g % 8) * 16
                cur = idxrow[tt_l, pl.ds(l0, 16)]
                t = tt_l * 128 + l0 + iota16
                pt = jnp.maximum(t - 1, 0)
                prev = plsc.load_gather(
                    idxrow, [lax.shift_right_logical(pt, 7), pt & 127])
                valid = (cur != 0) & ((cur != prev) | (t == 0))
                valrow[tt_l, pl.ds(l0, 16)] = valid.astype(jnp.int32)

        pltpu.sync_copy(idxrow, idx_hbm.at[bt, :, bs, :])
        pltpu.sync_copy(valrow, val_hbm.at[bt, :, bs, :])
        pltpu.sync_copy(mprow, mp_hbm.at[bt, :, bs, :])

    return k(lp5)


def _tc_body(x_ref, idx_ref, val_ref, mp_ref):
    cmax, cidx = _argmax_tree([x_ref[v] for v in range(V)])
    idx_ref[...] = cidx
    mp_ref[...] = jnp.exp(cmax)
    neg1 = jnp.full((8, 1), -1, jnp.int32)
    prev = jnp.concatenate([neg1, cidx[:, :-1]], axis=1)
    val_ref[...] = ((cidx != 0) & (cidx != prev)).astype(jnp.int32)


@jax.jit
def _ctc_tc(lp3):
    return pl.pallas_call(
        _tc_body,
        grid=(NBT - NBT_SC,),
        in_specs=[pl.BlockSpec((V, 8, T), lambda i: (0, i + NBT_SC, 0))],
        out_specs=[
            pl.BlockSpec((8, T), lambda i: (i, 0)),
            pl.BlockSpec((8, T), lambda i: (i, 0)),
            pl.BlockSpec((8, T), lambda i: (i, 0)),
        ],
        out_shape=[
            jax.ShapeDtypeStruct((B - B_SC, T), jnp.int32),
            jax.ShapeDtypeStruct((B - B_SC, T), jnp.int32),
            jax.ShapeDtypeStruct((B - B_SC, T), jnp.float32),
        ],
        compiler_params=pltpu.CompilerParams(
            dimension_semantics=("arbitrary",)),
    )(lp3)


def kernel(log_probs):
    lp5 = log_probs.reshape(NBT, 8, NTT, 128, V).transpose(4, 0, 2, 1, 3)
    lp3 = log_probs.transpose(2, 0, 1)

    sc_idx4, sc_val4, sc_mp4 = _ctc_sc(lp5)
    tc_idx, tc_val, tc_mp = _ctc_tc(lp3)

    def unview(x4):
        return x4.transpose(0, 2, 1, 3).reshape(B_SC, T)

    idx = jnp.concatenate([unview(sc_idx4), tc_idx], axis=0)
    val = jnp.concatenate([unview(sc_val4), tc_val], axis=0)
    mp = jnp.concatenate([unview(sc_mp4), tc_mp], axis=0)
    return idx, val.astype(bool), mp


# speedup vs baseline: 2.5099x; 1.0744x over previous
import jax
import jax.numpy as jnp
from jax import lax
from jax.experimental import pallas as pl
from jax.experimental.pallas import tpu as pltpu
from jax.experimental.pallas import tpu_sc as plsc

B = 128
T = 2048
V = 29
NUM_CORES = 2
NUM_SUBCORES = 16
NW = NUM_CORES * NUM_SUBCORES
NBT = B // 8
NTT = T // 128
NBT_SC = NW // 8
B_SC = 8 * NBT_SC

_mesh = plsc.VectorSubcoreMesh(
    core_axis_name="c", subcore_axis_name="s",
    num_cores=NUM_CORES, num_subcores=NUM_SUBCORES,
)


def _argmax_tree(xs):
    level = []
    for i in range(V // 2):
        a, b = xs[2 * i], xs[2 * i + 1]
        gt = b > a
        level.append((jnp.where(gt, b, a),
                      jnp.where(gt, jnp.int32(2 * i + 1), jnp.int32(2 * i))))
    level.append((xs[V - 1], jnp.full(xs[0].shape, V - 1, jnp.int32)))
    while len(level) > 1:
        nxt = []
        for i in range(len(level) // 2):
            va, ia = level[2 * i]
            vb, ib = level[2 * i + 1]
            gt = vb > va
            nxt.append((jnp.where(gt, vb, va), jnp.where(gt, ib, ia)))
        if len(level) % 2:
            nxt.append(level[-1])
        level = nxt
    return level[0]


@jax.jit
def _ctc_sc(lp5):
    @pl.kernel(
        out_type=[
            jax.ShapeDtypeStruct((NBT_SC, NTT, 8, 128), jnp.int32),
            jax.ShapeDtypeStruct((NBT_SC, NTT, 8, 128), jnp.int32),
            jax.ShapeDtypeStruct((NBT_SC, NTT, 8, 128), jnp.float32),
        ],
        mesh=_mesh,
        compiler_params=pltpu.CompilerParams(needs_layout_passes=False),
        scratch_types=[
            pltpu.VMEM((V, NTT // 2, 128), jnp.float32),
            pltpu.VMEM((V, NTT // 2, 128), jnp.float32),
            pltpu.VMEM((NTT, 128), jnp.int32),
            pltpu.VMEM((NTT, 128), jnp.int32),
            pltpu.VMEM((NTT, 128), jnp.float32),
            pltpu.SemaphoreType.DMA,
            pltpu.SemaphoreType.DMA,
        ],
    )
    def k(lp_hbm, idx_hbm, val_hbm, mp_hbm,
          buf0, buf1, idxrow, valrow, mprow, sem0, sem1):
        wid = lax.axis_index("s") * NUM_CORES + lax.axis_index("c")
        iota16 = lax.iota(jnp.int32, 16)
        bufs = (buf0, buf1)
        sems = (sem0, sem1)
        bt, bs = wid // 8, wid % 8
        half_tt = NTT // 2

        def start(half):
            return pltpu.async_copy(
                lp_hbm.at[:, bt, pl.ds(half * half_tt, half_tt), bs, :],
                bufs[half], sems[half])

        handles = {0: start(0)}
        for half in range(2):
            buf = bufs[half]
            handles.pop(half).wait()
            if half == 0:
                handles[1] = start(1)

            @plsc.parallel_loop(0, half_tt * 8)
            def _pass1(g):
                tt_l = half * half_tt + g // 8
                l0 = (g % 8) * 16
                xs = [buf[v, g // 8, pl.ds(l0, 16)] for v in range(V)]
                cmax, cidx = _argmax_tree(xs)
                idxrow[tt_l, pl.ds(l0, 16)] = cidx
                mprow[tt_l, pl.ds(l0, 16)] = jnp.exp(cmax)

            @plsc.parallel_loop(0, half_tt * 8)
            def _pass2(g):
                tt_l = half * half_tt + g // 8
                l0 = (g % 8) * 16
                cur = idxrow[tt_l, pl.ds(l0, 16)]
                t = tt_l * 128 + l0 + iota16
                pt = jnp.maximum(t - 1, 0)
                prev = plsc.load_gather(
                    idxrow, [lax.shift_right_logical(pt, 7), pt & 127])
                valid = (cur != 0) & ((cur != prev) | (t == 0))
                valrow[tt_l, pl.ds(l0, 16)] = valid.astype(jnp.int32)

        pltpu.sync_copy(idxrow, idx_hbm.at[bt, :, bs, :])
        pltpu.sync_copy(valrow, val_hbm.at[bt, :, bs, :])
        pltpu.sync_copy(mprow, mp_hbm.at[bt, :, bs, :])

    return k(lp5)


def _tc_body(x_ref, idx_ref, val_ref, mp_ref):
    cmax, cidx = _argmax_tree([x_ref[v] for v in range(V)])
    idx_ref[...] = cidx
    mp_ref[...] = jnp.exp(cmax)
    neg1 = jnp.full((8, 1), -1, jnp.int32)
    prev = jnp.concatenate([neg1, cidx[:, :-1]], axis=1)
    val_ref[...] = ((cidx != 0) & (cidx != prev)).astype(jnp.int32)


@jax.jit
def _ctc_tc(lp3):
    return pl.pallas_call(
        _tc_body,
        grid=(NBT - NBT_SC,),
        in_specs=[pl.BlockSpec((V, 8, T), lambda i: (0, i + NBT_SC, 0))],
        out_specs=[
            pl.BlockSpec((8, T), lambda i: (i + NBT_SC, 0)),
            pl.BlockSpec((8, T), lambda i: (i + NBT_SC, 0)),
            pl.BlockSpec((8, T), lambda i: (i + NBT_SC, 0)),
        ],
        out_shape=[
            jax.ShapeDtypeStruct((B, T), jnp.int32),
            jax.ShapeDtypeStruct((B, T), jnp.int32),
            jax.ShapeDtypeStruct((B, T), jnp.float32),
        ],
        compiler_params=pltpu.CompilerParams(
            dimension_semantics=("arbitrary",)),
    )(lp3)


def kernel(log_probs):
    lp5 = log_probs.reshape(NBT, 8, NTT, 128, V).transpose(4, 0, 2, 1, 3)
    lp3 = log_probs.transpose(2, 0, 1)

    sc_idx4, sc_val4, sc_mp4 = _ctc_sc(lp5)
    tc_idx, tc_val, tc_mp = _ctc_tc(lp3)

    def unview(x4):
        return x4.transpose(0, 2, 1, 3).reshape(B_SC, T)

    idx = lax.dynamic_update_slice(tc_idx, unview(sc_idx4), (0, 0))
    val = lax.dynamic_update_slice(tc_val, unview(sc_val4), (0, 0))
    mp = lax.dynamic_update_slice(tc_mp, unview(sc_mp4), (0, 0))
    return idx, val.astype(bool), mp
